# baseline (speedup 1.0000x reference)
import numpy as np
import jax
import jax.numpy as jnp
from jax import lax
from jax.experimental import pallas as pl
from jax.experimental.pallas import tpu as pltpu, tpu_sc as plsc

VOCAB = 100000
EMBED = 128
MAX_LEN = 512
SEQ_LEN = 200
BATCH = 1024
N_ROWS = BATCH * SEQ_LEN
N_SEG = 3

NUM_CORES = 2
NUM_SUBCORES = 16
NW = NUM_CORES * NUM_SUBCORES
ROWS_PER_W = N_ROWS // NW
CHUNK = 80
N_CHUNKS = ROWS_PER_W // CHUNK
NBUF = 5
AHEAD = 3
N_QUADS = N_CHUNKS // NBUF
GRP = 16
N_GRP = CHUNK // GRP


def _positional_table():
    pos = np.arange(MAX_LEN, dtype=np.float32)[:, None]
    div = np.exp(
        np.arange(0, EMBED, 2, dtype=np.float32) * -(np.log(10000.0) / EMBED))
    pe = np.zeros((MAX_LEN, EMBED), dtype=np.float32)
    pe[:, 0::2] = np.sin(pos * div)
    pe[:, 1::2] = np.cos(pos * div)
    return pe[:SEQ_LEN]


_PE = _positional_table()


def _embed_kernel(seq_hbm, lab_hbm, tok_hbm, comb_hbm, out_hbm, *rest):
    idx_vs = rest[0:NBUF]
    lab_v, rows_v, comb_v = rest[NBUF:NBUF + 3]
    sems = rest[NBUF + 3:]
    idx_sems = sems[0:NBUF]
    lab_sems = sems[NBUF:2 * NBUF]
    g_sems = sems[2 * NBUF:3 * NBUF]
    o_sems = sems[3 * NBUF:4 * NBUF]
    s_sem = sems[4 * NBUF]

    wid = lax.axis_index("s") * NUM_CORES + lax.axis_index("c")
    base = wid * ROWS_PER_W

    pltpu.async_copy(comb_hbm, comb_v, s_sem).wait()

    def fetch(g, buf):
        start = base + g * CHUNK
        pltpu.async_copy(
            seq_hbm.at[pl.ds(start, CHUNK)], idx_vs[buf], idx_sems[buf])
        pltpu.async_copy(
            lab_hbm.at[pl.ds(start, CHUNK)],
            lab_v.at[buf, pl.ds(0, CHUNK)], lab_sems[buf])

    def wait_fetch_idx(buf):
        pltpu.make_async_copy(
            seq_hbm.at[pl.ds(0, CHUNK)], idx_vs[buf], idx_sems[buf]).wait()

    def wait_fetch_lab(buf):
        pltpu.make_async_copy(
            lab_hbm.at[pl.ds(0, CHUNK)],
            lab_v.at[buf, pl.ds(0, CHUNK)], lab_sems[buf]).wait()

    def gather(buf):
        pltpu.async_copy(
            tok_hbm.at[idx_vs[buf]], rows_v.at[buf], g_sems[buf])

    def wait_gather(buf):
        pltpu.make_async_copy(
            tok_hbm.at[idx_vs[buf]], rows_v.at[buf], g_sems[buf]).wait()

    def put(buf, start):
        pltpu.async_copy(
            rows_v.at[buf], out_hbm.at[pl.ds(start, CHUNK)], o_sems[buf])

    def wait_put(buf):
        pltpu.make_async_copy(
            rows_v.at[buf], out_hbm.at[pl.ds(0, CHUNK)], o_sems[buf]).wait()

    def compute(buf, start):
        rv = rows_v.at[buf]
        lv = lab_v.at[buf]
        lpos0 = lax.rem(start, SEQ_LEN)
        lane = lax.iota(jnp.int32, 16)

        def grp_body(t, _):
            j0 = t * GRP
            labs = lv[pl.ds(j0, 16)]
            lpos_vec = lax.rem(lpos0 + j0 + lane, SEQ_LEN)
            sidx_vec = labs + SEQ_LEN
            for k in range(GRP):
                j = j0 + k
                pidx = lpos_vec[k]
                sidx = sidx_vec[k]
                pes = [comb_v[pidx, pl.ds(c * 16, 16)] for c in range(8)]
                sgs = [comb_v[sidx, pl.ds(c * 16, 16)] for c in range(8)]
                for c in range(EMBED // 16):
                    plsc.addupdate(rv.at[j, pl.ds(c * 16, 16)],
                                   pes[c] + sgs[c])
            return 0

        lax.fori_loop(0, N_GRP, grp_body, 0)

    for b in range(NBUF):
        fetch(b, b)
    for b in range(AHEAD):
        wait_fetch_idx(b)
        gather(b)

    def quad_body(q, _):
        for b in range(NBUF):
            g = NBUF * q + b
            start = base + g * CHUNK

            wait_gather(b)
            wait_fetch_lab(b)
            compute(b, start)

            @pl.when(g + NBUF < N_CHUNKS)
            def _():
                fetch(g + NBUF, b)

            put(b, start)

            nb = (b + AHEAD) % NBUF

            @pl.when(g + AHEAD < N_CHUNKS)
            def _():
                wait_fetch_idx(nb)

                @pl.when(g >= NBUF - AHEAD)
                def _():
                    wait_put(nb)

                gather(nb)

        return 0

    lax.fori_loop(0, N_QUADS, quad_body, 0)

    for b in range(NBUF):
        wait_put(b)


def kernel(sequence, segment_label, token_table, segment_table):
    seq_flat = sequence.reshape(-1).astype(jnp.int32)
    lab_flat = segment_label.reshape(-1).astype(jnp.int32)
    pe = jnp.asarray(_PE)
    comb = jnp.concatenate([pe, segment_table], axis=0)

    mesh = plsc.VectorSubcoreMesh(core_axis_name="c", subcore_axis_name="s")
    run = pl.kernel(
        _embed_kernel,
        mesh=mesh,
        out_type=jax.ShapeDtypeStruct((N_ROWS, EMBED), jnp.float32),
        scratch_types=(
            [pltpu.VMEM((CHUNK,), jnp.int32)] * NBUF
            + [
                pltpu.VMEM((NBUF, CHUNK), jnp.int32),
                pltpu.VMEM((NBUF, CHUNK, EMBED), jnp.float32),
                pltpu.VMEM((SEQ_LEN + N_SEG, EMBED), jnp.float32),
            ]
            + [pltpu.SemaphoreType.DMA] * (4 * NBUF + 1)),
    )
    out = run(seq_flat, lab_flat, token_table, comb)
    return out.reshape(BATCH, SEQ_LEN, EMBED)


# speedup vs baseline: 1.2532x; 1.2532x over previous
import numpy as np
import jax
import jax.numpy as jnp
from jax import lax
from jax.experimental import pallas as pl
from jax.experimental.pallas import tpu as pltpu, tpu_sc as plsc

VOCAB = 100000
EMBED = 128
MAX_LEN = 512
SEQ_LEN = 200
BATCH = 1024
N_ROWS = BATCH * SEQ_LEN
N_SEG = 3

NUM_CORES = 2
NUM_SUBCORES = 16
NW = NUM_CORES * NUM_SUBCORES
ROWS_PER_W = N_ROWS // NW
CHUNK = 80
N_CHUNKS = ROWS_PER_W // CHUNK
NBUF = 5
AHEAD = 3
N_QUADS = N_CHUNKS // NBUF
GRP = 16
N_GRP = CHUNK // GRP


def _positional_table():
    pos = np.arange(MAX_LEN, dtype=np.float32)[:, None]
    div = np.exp(
        np.arange(0, EMBED, 2, dtype=np.float32) * -(np.log(10000.0) / EMBED))
    pe = np.zeros((MAX_LEN, EMBED), dtype=np.float32)
    pe[:, 0::2] = np.sin(pos * div)
    pe[:, 1::2] = np.cos(pos * div)
    return pe[:SEQ_LEN]


_PE = _positional_table()


def _embed_kernel(seq_hbm, lab_hbm, tok_hbm, comb_hbm, out_hbm, *rest):
    idx_vs = rest[0:NBUF]
    lab_v, rows_v, comb_v = rest[NBUF:NBUF + 3]
    sems = rest[NBUF + 3:]
    idx_sems = sems[0:NBUF]
    lab_sems = sems[NBUF:2 * NBUF]
    g_sems = sems[2 * NBUF:3 * NBUF]
    o_sems = sems[3 * NBUF:4 * NBUF]
    s_sem = sems[4 * NBUF]

    wid = lax.axis_index("s") * NUM_CORES + lax.axis_index("c")
    base = wid * ROWS_PER_W

    pltpu.async_copy(comb_hbm, comb_v, s_sem).wait()

    def fetch(g, buf):
        start = base + g * CHUNK
        pltpu.async_copy(
            seq_hbm.at[pl.ds(start, CHUNK)], idx_vs[buf], idx_sems[buf])
        pltpu.async_copy(
            lab_hbm.at[pl.ds(start, CHUNK)],
            lab_v.at[buf, pl.ds(0, CHUNK)], lab_sems[buf])

    def wait_fetch_idx(buf):
        pltpu.make_async_copy(
            seq_hbm.at[pl.ds(0, CHUNK)], idx_vs[buf], idx_sems[buf]).wait()

    def wait_fetch_lab(buf):
        pltpu.make_async_copy(
            lab_hbm.at[pl.ds(0, CHUNK)],
            lab_v.at[buf, pl.ds(0, CHUNK)], lab_sems[buf]).wait()

    def gather(buf):
        pltpu.async_copy(
            tok_hbm.at[idx_vs[buf]], rows_v.at[buf], g_sems[buf])

    def wait_gather(buf):
        pltpu.make_async_copy(
            tok_hbm.at[idx_vs[buf]], rows_v.at[buf], g_sems[buf]).wait()

    def put(buf, start):
        pltpu.async_copy(
            rows_v.at[buf], out_hbm.at[pl.ds(start, CHUNK)], o_sems[buf])

    def wait_put(buf):
        pltpu.make_async_copy(
            rows_v.at[buf], out_hbm.at[pl.ds(0, CHUNK)], o_sems[buf]).wait()

    def compute(buf, start):
        rv = rows_v.at[buf]
        lv = lab_v.at[buf]
        lpos0 = lax.rem(start, SEQ_LEN)
        lane = lax.iota(jnp.int32, 16)

        def grp_body(t, _):
            j0 = t * GRP
            labs = lv[pl.ds(j0, 16)]
            lpos_vec = lax.rem(lpos0 + j0 + lane, SEQ_LEN)
            cidx_vec = labs * SEQ_LEN + lpos_vec
            for k in range(GRP):
                j = j0 + k
                cidx = cidx_vec[k]
                cmbs = [comb_v[cidx, pl.ds(c * 16, 16)] for c in range(8)]
                for c in range(EMBED // 16):
                    plsc.addupdate(rv.at[j, pl.ds(c * 16, 16)], cmbs[c])
            return 0

        lax.fori_loop(0, N_GRP, grp_body, 0)

    for b in range(NBUF):
        fetch(b, b)
    for b in range(AHEAD):
        wait_fetch_idx(b)
        gather(b)

    def quad_body(q, _):
        for b in range(NBUF):
            g = NBUF * q + b
            start = base + g * CHUNK

            wait_gather(b)
            wait_fetch_lab(b)
            compute(b, start)

            @pl.when(g + NBUF < N_CHUNKS)
            def _():
                fetch(g + NBUF, b)

            put(b, start)

            nb = (b + AHEAD) % NBUF

            @pl.when(g + AHEAD < N_CHUNKS)
            def _():
                wait_fetch_idx(nb)

                @pl.when(g >= NBUF - AHEAD)
                def _():
                    wait_put(nb)

                gather(nb)

        return 0

    lax.fori_loop(0, N_QUADS, quad_body, 0)

    for b in range(NBUF):
        wait_put(b)


def kernel(sequence, segment_label, token_table, segment_table):
    seq_flat = sequence.reshape(-1).astype(jnp.int32)
    lab_flat = segment_label.reshape(-1).astype(jnp.int32)
    pe = jnp.asarray(_PE)
    comb = (segment_table[:, None, :] + pe[None, :, :]).reshape(
        N_SEG * SEQ_LEN, EMBED)

    mesh = plsc.VectorSubcoreMesh(core_axis_name="c", subcore_axis_name="s")
    run = pl.kernel(
        _embed_kernel,
        mesh=mesh,
        out_type=jax.ShapeDtypeStruct((N_ROWS, EMBED), jnp.float32),
        scratch_types=(
            [pltpu.VMEM((CHUNK,), jnp.int32)] * NBUF
            + [
                pltpu.VMEM((NBUF, CHUNK), jnp.int32),
                pltpu.VMEM((NBUF, CHUNK, EMBED), jnp.float32),
                pltpu.VMEM((N_SEG * SEQ_LEN, EMBED), jnp.float32),
            ]
            + [pltpu.SemaphoreType.DMA] * (4 * NBUF + 1)),
    )
    out = run(seq_flat, lab_flat, token_table, comb)
    return out.reshape(BATCH, SEQ_LEN, EMBED)
